# Initial kernel scaffold; baseline (speedup 1.0000x reference)
import jax
import jax.numpy as jnp
from jax.experimental import pallas as pl


def kernel(x, edge_index, W1, b1, W2, b2):
    raise NotImplementedError("write your pallas kernel here")


# speedup vs baseline: 22.6051x; 22.6051x over previous
import functools

import jax
import jax.numpy as jnp
from jax import lax
from jax.experimental import pallas as pl
from jax.experimental.pallas import tpu as pltpu
from jax.experimental.pallas import tpu_sc as plsc

NC = 2
NS = 16
NW = NC * NS
CH = 128
LANES = 16
D = 128


def _sc_mesh():
    return plsc.VectorSubcoreMesh(core_axis_name="c", subcore_axis_name="s")


def _make_deg_kernel(n_chunks, r_pad):
    rpt = r_pad // NS
    rv = ((rpt + LANES - 1) // LANES) * LANES

    @functools.partial(
        pl.kernel,
        out_type=jax.ShapeDtypeStruct((NC * r_pad,), jnp.float32),
        mesh=_sc_mesh(),
        scratch_types=[
            pltpu.VMEM((n_chunks, CH), jnp.int32),
            pltpu.VMEM((CH,), jnp.float32),
            pltpu.VMEM((rv,), jnp.float32),
            pltpu.VMEM_SHARED((r_pad,), jnp.float32),
        ],
    )
    def deg_kernel(dst_hbm, out_hbm, dst_v, ones_v, stage_v, deg_sh):
        c = lax.axis_index("c")
        s = lax.axis_index("s")
        wid = c * NS + s
        for i in range(CH // LANES):
            ones_v[pl.ds(i * LANES, LANES)] = jnp.ones((LANES,), jnp.float32)
        for i in range(rv // LANES):
            stage_v[pl.ds(i * LANES, LANES)] = jnp.zeros((LANES,), jnp.float32)
        pltpu.sync_copy(stage_v.at[pl.ds(0, rpt)],
                        deg_sh.at[pl.ds(s * rpt, rpt)])
        pltpu.sync_copy(dst_hbm.at[wid], dst_v)
        plsc.subcore_barrier()

        def body(j, carry):
            pltpu.sync_copy(ones_v, deg_sh.at[dst_v.at[j]], add=True)
            return carry

        lax.fori_loop(0, n_chunks, body, 0)
        plsc.subcore_barrier()
        pltpu.sync_copy(deg_sh.at[pl.ds(s * rpt, rpt)],
                        stage_v.at[pl.ds(0, rpt)])
        pltpu.sync_copy(stage_v.at[pl.ds(0, rpt)],
                        out_hbm.at[pl.ds(c * r_pad + s * rpt, rpt)])

    return deg_kernel


def _make_row_scatter_kernel(n_chunks, r_pad):
    rpt = r_pad // NS

    @functools.partial(
        pl.kernel,
        out_type=jax.ShapeDtypeStruct((NC, r_pad, D), jnp.float32),
        mesh=_sc_mesh(),
        scratch_types=[
            pltpu.VMEM((n_chunks, CH), jnp.int32),
            pltpu.VMEM((n_chunks, CH), jnp.int32),
            pltpu.VMEM((CH, D), jnp.float32),
            pltpu.VMEM_SHARED((r_pad, D), jnp.float32),
            pltpu.SemaphoreType.DMA,
        ],
    )
    def scat_kernel(y_hbm, src_hbm, dst_hbm, zeros_hbm, out_hbm,
                    src_v, dst_v, rows_v, acc_sh, sem):
        c = lax.axis_index("c")
        s = lax.axis_index("s")
        wid = c * NS + s
        pltpu.sync_copy(zeros_hbm.at[pl.ds(s * rpt, rpt)],
                        acc_sh.at[pl.ds(s * rpt, rpt)])
        pltpu.sync_copy(src_hbm.at[wid], src_v)
        pltpu.sync_copy(dst_hbm.at[wid], dst_v)
        plsc.subcore_barrier()

        def body(j, carry):
            pltpu.async_copy(y_hbm.at[src_v.at[j]], rows_v, sem).wait()
            pltpu.sync_copy(rows_v, acc_sh.at[dst_v.at[j]], add=True)
            return carry

        lax.fori_loop(0, n_chunks, body, 0)
        plsc.subcore_barrier()
        pltpu.sync_copy(acc_sh.at[pl.ds(s * rpt, rpt)],
                        out_hbm.at[c, pl.ds(s * rpt, rpt)])

    return scat_kernel


def _tc1_body(x_ref, w_ref, dinv_ref, y_ref):
    xw = jnp.dot(x_ref[...], w_ref[...], preferred_element_type=jnp.float32)
    y_ref[...] = xw * dinv_ref[...]


def _tc2_body(acc_ref, y1_ref, dinv_ref, w_ref, b_ref, y2_ref):
    s = acc_ref[0] + acc_ref[1] + y1_ref[...]
    h = jnp.maximum(s * dinv_ref[...] + b_ref[...], 0.0)
    xw = jnp.dot(h, w_ref[...], preferred_element_type=jnp.float32)
    y2_ref[...] = xw * dinv_ref[...]


def _tc3_body(acc_ref, y2_ref, dinv_ref, b_ref, o_ref):
    o = (acc_ref[0] + acc_ref[1] + y2_ref[...]) * dinv_ref[...] + b_ref[...]
    m = jnp.max(o, axis=1, keepdims=True)
    lse = jnp.log(jnp.sum(jnp.exp(o - m), axis=1, keepdims=True)) + m
    o_ref[...] = o - lse


def _tc1(x, w, dinv, bm):
    n = x.shape[0]
    return pl.pallas_call(
        _tc1_body,
        grid=(n // bm,),
        in_specs=[
            pl.BlockSpec((bm, D), lambda i: (i, 0)),
            pl.BlockSpec((D, D), lambda i: (0, 0)),
            pl.BlockSpec((bm, 1), lambda i: (i, 0)),
        ],
        out_specs=pl.BlockSpec((bm, D), lambda i: (i, 0)),
        out_shape=jax.ShapeDtypeStruct((n, D), jnp.float32),
    )(x, w, dinv)


def _tc2(acc, y1, dinv, w, b, bm):
    n = y1.shape[0]
    return pl.pallas_call(
        _tc2_body,
        grid=(n // bm,),
        in_specs=[
            pl.BlockSpec((NC, bm, D), lambda i: (0, i, 0)),
            pl.BlockSpec((bm, D), lambda i: (i, 0)),
            pl.BlockSpec((bm, 1), lambda i: (i, 0)),
            pl.BlockSpec((D, D), lambda i: (0, 0)),
            pl.BlockSpec((1, D), lambda i: (0, 0)),
        ],
        out_specs=pl.BlockSpec((bm, D), lambda i: (i, 0)),
        out_shape=jax.ShapeDtypeStruct((n, D), jnp.float32),
    )(acc, y1, dinv, w, b)


def _tc3(acc, y2, dinv, b, bm):
    n = y2.shape[0]
    return pl.pallas_call(
        _tc3_body,
        grid=(n // bm,),
        in_specs=[
            pl.BlockSpec((NC, bm, D), lambda i: (0, i, 0)),
            pl.BlockSpec((bm, D), lambda i: (i, 0)),
            pl.BlockSpec((bm, 1), lambda i: (i, 0)),
            pl.BlockSpec((1, D), lambda i: (0, 0)),
        ],
        out_specs=pl.BlockSpec((bm, D), lambda i: (i, 0)),
        out_shape=jax.ShapeDtypeStruct((n, D), jnp.float32),
    )(acc, y2, dinv, b)


def kernel(x, edge_index, W1, b1, W2, b2):
    n = x.shape[0]
    e = edge_index.shape[1]
    bm = 1000

    r_pad = ((n + 8 + 127) // 128) * 128

    n_chunks = -(-e // (NW * CH))
    n_chunks += n_chunks % 2
    e_pad = NW * CH * n_chunks
    src = edge_index[0].astype(jnp.int32)
    dst = edge_index[1].astype(jnp.int32)
    pad = e_pad - e
    if pad:
        ar = jnp.arange(pad, dtype=jnp.int32)
        src = jnp.concatenate([src, ar % jnp.int32(min(n, 512))])
        dst = jnp.concatenate([dst, n + (ar % 8)])
    src3 = src.reshape(NW, n_chunks, CH)
    dst3 = dst.reshape(NW, n_chunks, CH)

    z2 = jnp.zeros((r_pad, D), jnp.float32)

    deg_p = _make_deg_kernel(n_chunks, r_pad)(dst3).reshape(NC, r_pad)
    deg = deg_p[0, :n] + deg_p[1, :n] + 1.0
    dinv = lax.rsqrt(deg).reshape(n, 1)

    scat = _make_row_scatter_kernel(n_chunks, r_pad)

    y1 = _tc1(x, W1, dinv, bm)
    acc1 = scat(y1, src3, dst3, z2)
    y2 = _tc2(acc1, y1, dinv, W2, b1.reshape(1, D), bm)
    acc2 = scat(y2, src3, dst3, z2)
    return _tc3(acc2, y2, dinv, b2.reshape(1, D), bm)
